# baseline (speedup 1.0000x reference)
import functools

import jax
import jax.numpy as jnp
from jax import lax
from jax.experimental import pallas as pl
from jax.experimental.pallas import tpu as pltpu
from jax.experimental.pallas import tpu_sc as plsc

BATCH = 16384
NF = 32
MD = 128
NC = 2
NS = 16
NW = NC * NS
BPW = BATCH // NW
CHUNK = 128
NCH = BPW // CHUNK

@functools.cache
def _make_sc_gather():
    mesh = plsc.VectorSubcoreMesh(core_axis_name="c", subcore_axis_name="s")

    @functools.partial(
        pl.kernel,
        out_type=(
            jax.ShapeDtypeStruct((BATCH, NF), jnp.float32),
            jax.ShapeDtypeStruct((BATCH, NF), jnp.float32),
            jax.ShapeDtypeStruct((BATCH, MD), jnp.float32),
            jax.ShapeDtypeStruct((BATCH, MD), jnp.float32),
        ),
        mesh=mesh,
        scratch_types=[
            pltpu.VMEM((NCH, CHUNK), jnp.int32),
            pltpu.VMEM((NCH, CHUNK), jnp.int32),
            pltpu.VMEM((BPW, NF), jnp.float32),
            pltpu.VMEM((BPW, NF), jnp.float32),
            pltpu.VMEM((BPW, MD), jnp.float32),
            pltpu.SemaphoreType.DMA,
            pltpu.SemaphoreType.DMA,
        ],
        compiler_params=pltpu.CompilerParams(use_tc_tiling_on_sc=False),
    )
    def _sc_gather(user_h, item_h, gue_h, gie_h, mue_h, mie_h,
                   gu_out, gi_out, mu_out, mi_out,
                   uidx_v, iidx_v, gu_v, gi_v, m_v, sem_g, sem_m):
        wid = lax.axis_index("s") * NC + lax.axis_index("c")
        base = wid * BPW
        for j in range(NCH):
            pltpu.sync_copy(user_h.at[pl.ds(base + j * CHUNK, CHUNK)],
                            uidx_v.at[j])
            pltpu.sync_copy(item_h.at[pl.ds(base + j * CHUNK, CHUNK)],
                            iidx_v.at[j])
        gcps = []
        for j in range(NCH):
            gcps.append(pltpu.async_copy(
                gue_h.at[uidx_v.at[j]], gu_v.at[pl.ds(j * CHUNK, CHUNK)], sem_g))
            gcps.append(pltpu.async_copy(
                gie_h.at[iidx_v.at[j]], gi_v.at[pl.ds(j * CHUNK, CHUNK)], sem_g))
        mcps = []
        for j in range(NCH):
            mcps.append(pltpu.async_copy(
                mue_h.at[uidx_v.at[j]], m_v.at[pl.ds(j * CHUNK, CHUNK)], sem_m))
        for cp in gcps:
            cp.wait()
        pltpu.sync_copy(gu_v, gu_out.at[pl.ds(base, BPW)])
        pltpu.sync_copy(gi_v, gi_out.at[pl.ds(base, BPW)])
        for cp in mcps:
            cp.wait()
        pltpu.sync_copy(m_v, mu_out.at[pl.ds(base, BPW)])
        mcps = []
        for j in range(NCH):
            mcps.append(pltpu.async_copy(
                mie_h.at[iidx_v.at[j]], m_v.at[pl.ds(j * CHUNK, CHUNK)], sem_m))
        for cp in mcps:
            cp.wait()
        pltpu.sync_copy(m_v, mi_out.at[pl.ds(base, BPW)])

    return _sc_gather


TB = 2048


def _tc_body(gu, gi, mu, mi, w1a, w1b, b1, w2, b2, w3, b3, wpg, wpx, bp, out):
    f32 = jnp.float32
    hp = lax.Precision.HIGHEST
    x1 = (jnp.dot(mu[...], w1a[...], precision=hp, preferred_element_type=f32)
          + jnp.dot(mi[...], w1b[...], precision=hp, preferred_element_type=f32)
          + b1[...])
    h1 = jnp.maximum(x1, 0.0)
    h2 = jnp.maximum(
        jnp.dot(h1, w2[...], precision=hp, preferred_element_type=f32) + b2[...], 0.0)
    h3 = jnp.maximum(
        jnp.dot(h2, w3[...], precision=hp, preferred_element_type=f32) + b3[...], 0.0)
    gmf = gu[...] * gi[...]
    pred = (jnp.sum(gmf * wpg[...], axis=1)
            + jnp.sum(h3 * wpx[...], axis=1) + bp[0, 0])
    out[...] = pred


def _rep(shape):
    return pl.BlockSpec(shape, lambda i: tuple(0 for _ in shape))


_tc_call = pl.pallas_call(
    _tc_body,
    grid=(BATCH // TB,),
    in_specs=[
        pl.BlockSpec((TB, NF), lambda i: (i, 0)),
        pl.BlockSpec((TB, NF), lambda i: (i, 0)),
        pl.BlockSpec((TB, MD), lambda i: (i, 0)),
        pl.BlockSpec((TB, MD), lambda i: (i, 0)),
        _rep((MD, MD)),
        _rep((MD, MD)),
        _rep((1, MD)),
        _rep((MD, MD // 2)),
        _rep((1, MD // 2)),
        _rep((MD // 2, NF)),
        _rep((1, NF)),
        _rep((1, NF)),
        _rep((1, NF)),
        _rep((1, 1)),
    ],
    out_specs=pl.BlockSpec((TB,), lambda i: (i,)),
    out_shape=jax.ShapeDtypeStruct((BATCH,), jnp.float32),
)


def kernel(user, item, gmf_user_emb, gmf_item_emb, mlp_user_emb, mlp_item_emb,
           W1, b1, W2, b2, W3, b3, Wp, bp):
    user = user.astype(jnp.int32)
    item = item.astype(jnp.int32)
    gu, gi, mu, mi = _make_sc_gather()(user, item, gmf_user_emb, gmf_item_emb,
                                       mlp_user_emb, mlp_item_emb)
    w1a, w1b = W1[:MD], W1[MD:]
    wpg = Wp[:NF].reshape(1, NF)
    wpx = Wp[NF:].reshape(1, NF)
    return _tc_call(gu, gi, mu, mi, w1a, w1b, b1.reshape(1, MD),
                    W2, b2.reshape(1, MD // 2), W3, b3.reshape(1, NF),
                    wpg, wpx, bp.reshape(1, 1))


# speedup vs baseline: 1.4075x; 1.4075x over previous
import functools

import jax
import jax.numpy as jnp
from jax import lax
from jax.experimental import pallas as pl
from jax.experimental.pallas import tpu as pltpu
from jax.experimental.pallas import tpu_sc as plsc

BATCH = 16384
NF = 32
MD = 128
NROWS = 100000
NC = 2
NS = 16
NW = NC * NS
BPW = BATCH // NW
CHUNK = 128
NCH = BPW // CHUNK

DT_COLS = 2048
DT_GRID = -(-NROWS // DT_COLS)


def _detile_body(xt_ref, o_ref):
    o_ref[...] = pltpu.einshape("k(qj)->q(jk)", xt_ref[...], j=4)


_detile = pl.pallas_call(
    _detile_body,
    grid=(DT_GRID,),
    in_specs=[pl.BlockSpec((NF, DT_COLS), lambda i: (0, i))],
    out_specs=pl.BlockSpec((DT_COLS // 4, 128), lambda i: (i, 0)),
    out_shape=jax.ShapeDtypeStruct((NROWS // 4, 128), jnp.float32),
)


@functools.cache
def _make_sc_mlp_gather():
    mesh = plsc.VectorSubcoreMesh(core_axis_name="c", subcore_axis_name="s")

    @functools.partial(
        pl.kernel,
        out_type=(
            jax.ShapeDtypeStruct((BATCH, MD), jnp.float32),
            jax.ShapeDtypeStruct((BATCH, MD), jnp.float32),
        ),
        mesh=mesh,
        scratch_types=[
            pltpu.VMEM((NCH, CHUNK), jnp.int32),
            pltpu.VMEM((NCH, CHUNK), jnp.int32),
            pltpu.VMEM((BPW, MD), jnp.float32),
            pltpu.SemaphoreType.DMA,
        ],
        compiler_params=pltpu.CompilerParams(use_tc_tiling_on_sc=False),
    )
    def _sc_mlp(user_h, item_h, mue_h, mie_h, mu_out, mi_out,
                uidx_v, iidx_v, m_v, sem):
        wid = lax.axis_index("s") * NC + lax.axis_index("c")
        base = wid * BPW
        for j in range(NCH):
            pltpu.sync_copy(user_h.at[pl.ds(base + j * CHUNK, CHUNK)],
                            uidx_v.at[j])
            pltpu.sync_copy(item_h.at[pl.ds(base + j * CHUNK, CHUNK)],
                            iidx_v.at[j])
        cps = []
        for j in range(NCH):
            cps.append(pltpu.async_copy(
                mue_h.at[uidx_v.at[j]], m_v.at[pl.ds(j * CHUNK, CHUNK)], sem))
        for cp in cps:
            cp.wait()
        pltpu.sync_copy(m_v, mu_out.at[pl.ds(base, BPW)])
        cps = []
        for j in range(NCH):
            cps.append(pltpu.async_copy(
                mie_h.at[iidx_v.at[j]], m_v.at[pl.ds(j * CHUNK, CHUNK)], sem))
        for cp in cps:
            cp.wait()
        pltpu.sync_copy(m_v, mi_out.at[pl.ds(base, BPW)])

    return _sc_mlp


@functools.cache
def _make_sc_gmf_gather():
    mesh = plsc.VectorSubcoreMesh(core_axis_name="c", subcore_axis_name="s")

    @functools.partial(
        pl.kernel,
        out_type=jax.ShapeDtypeStruct((BATCH, 4 * NF), jnp.float32),
        mesh=mesh,
        scratch_types=[
            pltpu.VMEM((NCH, CHUNK), jnp.int32),
            pltpu.VMEM((NCH, CHUNK), jnp.int32),
            pltpu.VMEM((BPW, NF), jnp.float32),
            pltpu.VMEM((BPW, NF), jnp.float32),
            pltpu.SemaphoreType.DMA,
        ],
        compiler_params=pltpu.CompilerParams(use_tc_tiling_on_sc=False),
    )
    def _sc_gmf(user_h, item_h, gue_h, gie_h, g_out,
                uidx_v, iidx_v, gu_v, gi_v, sem):
        wid = lax.axis_index("s") * NC + lax.axis_index("c")
        base = wid * BPW
        for j in range(NCH):
            pltpu.sync_copy(user_h.at[pl.ds(base + j * CHUNK, CHUNK)],
                            uidx_v.at[j])
            pltpu.sync_copy(item_h.at[pl.ds(base + j * CHUNK, CHUNK)],
                            iidx_v.at[j])
        cps = []
        for j in range(NCH):
            cps.append(pltpu.async_copy(
                gue_h.at[uidx_v.at[j]], gu_v.at[pl.ds(j * CHUNK, CHUNK)], sem))
            cps.append(pltpu.async_copy(
                gie_h.at[iidx_v.at[j]], gi_v.at[pl.ds(j * CHUNK, CHUNK)], sem))
        for cp in cps:
            cp.wait()
        pltpu.sync_copy(gu_v, g_out.at[pl.ds(base, BPW), pl.ds(0, NF)])
        pltpu.sync_copy(gi_v, g_out.at[pl.ds(base, BPW), pl.ds(NF, NF)])

    return _sc_gmf


TB = 2048


def _tc_body(g, mu, mi, w1a, w1b, b1, w2, b2, w3, b3, wpg, wpx, bp, out):
    f32 = jnp.float32
    gblk = g[...]
    x1 = (jnp.dot(mu[...], w1a[...], preferred_element_type=f32)
          + jnp.dot(mi[...], w1b[...], preferred_element_type=f32)
          + b1[...])
    h1 = jnp.maximum(x1, 0.0)
    h2 = jnp.maximum(
        jnp.dot(h1, w2[...], preferred_element_type=f32) + b2[...], 0.0)
    h3 = jnp.maximum(
        jnp.dot(h2, w3[...], preferred_element_type=f32) + b3[...], 0.0)
    gmf = gblk[:, :NF] * gblk[:, NF:2 * NF]
    pred = (jnp.sum(gmf * wpg[...], axis=1)
            + jnp.sum(h3 * wpx[...], axis=1) + bp[0, 0])
    out[...] = pred


def _rep(shape):
    return pl.BlockSpec(shape, lambda i: tuple(0 for _ in shape))


_tc_call = pl.pallas_call(
    _tc_body,
    grid=(BATCH // TB,),
    in_specs=[
        pl.BlockSpec((TB, 4 * NF), lambda i: (i, 0)),
        pl.BlockSpec((TB, MD), lambda i: (i, 0)),
        pl.BlockSpec((TB, MD), lambda i: (i, 0)),
        _rep((MD, MD)),
        _rep((MD, MD)),
        _rep((1, MD)),
        _rep((MD, MD // 2)),
        _rep((1, MD // 2)),
        _rep((MD // 2, NF)),
        _rep((1, NF)),
        _rep((1, NF)),
        _rep((1, NF)),
        _rep((1, 1)),
    ],
    out_specs=pl.BlockSpec((TB,), lambda i: (i,)),
    out_shape=jax.ShapeDtypeStruct((BATCH,), jnp.float32),
)


def kernel(user, item, gmf_user_emb, gmf_item_emb, mlp_user_emb, mlp_item_emb,
           W1, b1, W2, b2, W3, b3, Wp, bp):
    user = user.astype(jnp.int32)
    item = item.astype(jnp.int32)
    mu, mi = _make_sc_mlp_gather()(user, item, mlp_user_emb, mlp_item_emb)
    g = _make_sc_gmf_gather()(user, item, gmf_user_emb, gmf_item_emb)
    w1a, w1b = W1[:MD], W1[MD:]
    wpg = Wp[:NF].reshape(1, NF)
    wpx = Wp[NF:].reshape(1, NF)
    return _tc_call(g, mu, mi, w1a, w1b, b1.reshape(1, MD),
                    W2, b2.reshape(1, MD // 2), W3, b3.reshape(1, NF),
                    wpg, wpx, bp.reshape(1, 1))
